# baseline (speedup 1.0000x reference)
import functools

import jax
import jax.numpy as jnp
from jax import lax
from jax.experimental import pallas as pl
from jax.experimental.pallas import tpu as pltpu
from jax.experimental.pallas import tpu_sc as plsc

_NUM_CORES = 2
_NUM_SUBCORES = 16
_NW = _NUM_CORES * _NUM_SUBCORES


def _transpose_body(x_ref, eye_ref, o_ref):
    o_ref[...] = jax.lax.dot_general(
        x_ref[...],
        eye_ref[...],
        (((0,), (0,)), ((), ())),
        preferred_element_type=jnp.float32,
    )


def _transpose(x3):
    B, C, L = x3.shape
    S = 3584
    assert L % S == 0
    eye = jnp.eye(C, dtype=jnp.float32)
    return pl.pallas_call(
        _transpose_body,
        grid=(B, L // S),
        in_specs=[
            pl.BlockSpec((None, C, S), lambda b, k: (b, 0, k)),
            pl.BlockSpec((C, C), lambda b, k: (0, 0)),
        ],
        out_specs=pl.BlockSpec((None, S, C), lambda b, k: (b, k, 0)),
        out_shape=jax.ShapeDtypeStruct((B, L, C), x3.dtype),
    )(x3, eye)


def _make_sc_gather(B, L, C, nchunk, ch):
    mesh = plsc.VectorSubcoreMesh(
        core_axis_name="c",
        subcore_axis_name="s",
        num_cores=_NUM_CORES,
        num_subcores=_NUM_SUBCORES,
    )
    rpw = nchunk * ch

    @functools.partial(
        pl.kernel,
        out_type=jax.ShapeDtypeStruct((B, L, C), jnp.float32),
        mesh=mesh,
        compiler_params=pltpu.CompilerParams(use_tc_tiling_on_sc=False),
        scratch_types=[
            pltpu.VMEM((nchunk * ch,), jnp.int32),
            pltpu.VMEM((ch, C), jnp.float32),
            pltpu.VMEM((ch, C), jnp.float32),
            pltpu.SemaphoreType.DMA,
            pltpu.SemaphoreType.DMA,
        ],
    )
    def sc_gather(xt_hbm, idx_hbm, out_hbm, idx_v, buf0, buf1, sem0, sem1):
        wid = lax.axis_index("s") * _NUM_CORES + lax.axis_index("c")
        pltpu.sync_copy(idx_hbm.at[wid], idx_v)

        bufs = (buf0, buf1)
        sems = (sem0, sem1)
        base = wid * rpw
        prev = None
        for i in range(B * nchunk):
            b, k = divmod(i, nchunk)
            bi = i % 2
            cp = pltpu.async_copy(
                xt_hbm.at[b].at[idx_v.at[pl.ds(k * ch, ch)]], bufs[bi], sems[bi]
            )
            if prev is not None:
                pcp, pbi, pb, pk = prev
                pcp.wait()
                pltpu.sync_copy(
                    bufs[pbi], out_hbm.at[pb].at[pl.ds(base + pk * ch, ch)]
                )
            prev = (cp, bi, b, k)
        pcp, pbi, pb, pk = prev
        pcp.wait()
        pltpu.sync_copy(bufs[pbi], out_hbm.at[pb].at[pl.ds(base + pk * ch, ch)])

    return sc_gather


def kernel(x, hilbert_indices):
    B, C, H, W = x.shape
    L = H * W
    idx = hilbert_indices.astype(jnp.int32)

    xt = _transpose(x.reshape(B, C, L))

    nchunk, ch = 7, 224
    assert _NW * nchunk * ch == L
    idx_r = idx.reshape(_NW, nchunk * ch)
    return _make_sc_gather(B, L, C, nchunk, ch)(xt, idx_r)


# speedup vs baseline: 1.1394x; 1.1394x over previous
import functools

import jax
import jax.numpy as jnp
from jax import lax
from jax.experimental import pallas as pl
from jax.experimental.pallas import tpu as pltpu
from jax.experimental.pallas import tpu_sc as plsc

_NUM_CORES = 2
_NUM_SUBCORES = 16
_NW = _NUM_CORES * _NUM_SUBCORES


_HB = 8


def _transpose_body(x_ref, eye_ref, o_ref):
    W = x_ref.shape[2]
    for h in range(_HB):
        o_ref[pl.ds(h * W, W)] = jax.lax.dot_general(
            x_ref[:, h, :],
            eye_ref[...],
            (((0,), (0,)), ((), ())),
            preferred_element_type=jnp.float32,
        )


def _transpose(x):
    B, C, H, W = x.shape
    L = H * W
    eye = jnp.eye(C, dtype=jnp.float32)
    return pl.pallas_call(
        _transpose_body,
        grid=(B, H // _HB),
        in_specs=[
            pl.BlockSpec((None, C, _HB, W), lambda b, k: (b, 0, k, 0)),
            pl.BlockSpec((C, C), lambda b, k: (0, 0)),
        ],
        out_specs=pl.BlockSpec((None, _HB * W, C), lambda b, k: (b, k, 0)),
        out_shape=jax.ShapeDtypeStruct((B, L, C), x.dtype),
    )(x, eye)


def _make_sc_gather(B, L, C, nchunk, ch):
    mesh = plsc.VectorSubcoreMesh(
        core_axis_name="c",
        subcore_axis_name="s",
        num_cores=_NUM_CORES,
        num_subcores=_NUM_SUBCORES,
    )
    rpw = nchunk * ch

    @functools.partial(
        pl.kernel,
        out_type=jax.ShapeDtypeStruct((B, L, C), jnp.float32),
        mesh=mesh,
        compiler_params=pltpu.CompilerParams(use_tc_tiling_on_sc=False),
        scratch_types=[
            pltpu.VMEM((nchunk * ch,), jnp.int32),
            pltpu.VMEM((ch, C), jnp.float32),
            pltpu.VMEM((ch, C), jnp.float32),
            pltpu.SemaphoreType.DMA,
            pltpu.SemaphoreType.DMA,
        ],
    )
    def sc_gather(xt_hbm, idx_hbm, out_hbm, idx_v, buf0, buf1, sem0, sem1):
        wid = lax.axis_index("s") * _NUM_CORES + lax.axis_index("c")
        pltpu.sync_copy(idx_hbm.at[wid], idx_v)

        bufs = (buf0, buf1)
        sems = (sem0, sem1)
        base = wid * rpw
        prev = None
        for i in range(B * nchunk):
            b, k = divmod(i, nchunk)
            bi = i % 2
            cp = pltpu.async_copy(
                xt_hbm.at[b].at[idx_v.at[pl.ds(k * ch, ch)]], bufs[bi], sems[bi]
            )
            if prev is not None:
                pcp, pbi, pb, pk = prev
                pcp.wait()
                pltpu.sync_copy(
                    bufs[pbi], out_hbm.at[pb].at[pl.ds(base + pk * ch, ch)]
                )
            prev = (cp, bi, b, k)
        pcp, pbi, pb, pk = prev
        pcp.wait()
        pltpu.sync_copy(bufs[pbi], out_hbm.at[pb].at[pl.ds(base + pk * ch, ch)])

    return sc_gather


def kernel(x, hilbert_indices):
    B, C, H, W = x.shape
    L = H * W
    idx = hilbert_indices.astype(jnp.int32)

    xt = _transpose(x)

    nchunk, ch = 7, 224
    assert _NW * nchunk * ch == L
    idx_r = idx.reshape(_NW, nchunk * ch)
    return _make_sc_gather(B, L, C, nchunk, ch)(xt, idx_r)
